# baseline (speedup 1.0000x reference)
import jax
import jax.numpy as jnp
from jax import lax
from jax.experimental import pallas as pl
from jax.experimental.pallas import tpu as pltpu
from jax.experimental.pallas import tpu_sc as plsc

N_CENTER = 100000
D = 64
B = 16384
ALPHA = 0.9
A1 = ALPHA - 1.0

NC = 2
NS = 16
BT = B // NS
HALF = N_CENTER // NC
NCHUNK = 3
CHUNK = 16672
BLK = 64
NG = BT // 16
LCAP = BT + 16
TROWS = 1048
NDB = 17


def _body(feat_hbm, lab_hbm, ctr_hbm, o16_hbm, out_hbm,
          lab_v, idx_v, ones16_v, zbuf, cnt0,
          rel0, rel1, rel2, pos0, pos1, pos2,
          feat0, feat1, accb0, accb1, cntb0, cntb1,
          sem_a, sem_b, sem_c, sem_d, sem_e, sem_f, sem_oa, sem_ob,
          acc_sh, cnt_sh):
    c = lax.axis_index("c")
    s = lax.axis_index("s")

    pltpu.sync_copy(lab_hbm.at[s], lab_v)
    pltpu.sync_copy(o16_hbm, ones16_v)

    rel = [rel0, rel1, rel2]
    pos = [pos0, pos1, pos2]
    ctrb = [feat0, feat1]
    accb = [accb0, accb1]
    cntb = [cntb0, cntb1]
    asem = [sem_a, sem_b]
    bsem = [sem_c, sem_d]
    csem = [sem_e, sem_f]
    osem = [sem_oa, sem_ob]
    lanes = lax.iota(jnp.int32, 16)

    def fill_zb(i, _):
        zbuf[i // 4, pl.ds((i % 4) * 16, 16)] = jnp.zeros((16,), jnp.float32)
        return 0
    lax.fori_loop(0, BLK * 4, fill_zb, 0, unroll=4)

    def fill_c0(i, _):
        cnt0[i, pl.ds(0, 16)] = jnp.zeros((16,), jnp.float32)
        return 0
    lax.fori_loop(0, BLK, fill_c0, 0, unroll=4)

    nblk = []
    for q in range(NCHUNK):
        def prefill(i, _):
            rel[q][pl.ds(i * 16, 16)] = CHUNK + (i % 16) * 16 + lanes
            pos[q][pl.ds(i * 16, 16)] = jnp.zeros((16,), jnp.int32)
            return 0
        lax.fori_loop(0, LCAP // 16, prefill, 0, unroll=4)

        base = c * HALF + q * CHUNK

        def compact(i, off):
            v = lab_v[pl.ds(i * 16, 16)]
            r = v - base
            m = (r >= 0) & (r < CHUNK)
            mi = m.astype(jnp.int32)
            incl = plsc.cumsum(mi)
            dst = off + incl - mi
            plsc.store_scatter(rel[q], [dst], r, mask=m)
            p = s * BT + i * 16 + lanes
            plsc.store_scatter(pos[q], [dst], p, mask=m)
            return off + plsc.all_reduce_population_count(m)[0]
        off = lax.fori_loop(0, NG, compact, jnp.int32(0))
        nblk.append((off + (BLK - 1)) // BLK)

    for q in range(NCHUNK):
        base = c * HALF + q * CHUNK
        crows = min(CHUNK, HALF - q * CHUNK)
        tstart = jnp.minimum(s * TROWS, crows - TROWS)

        def dense_rb(b):
            return tstart + jnp.minimum(b * BLK, TROWS - BLK)

        zd = [pltpu.async_copy(cnt0, cnt_sh.at[pl.ds(dense_rb(b), BLK)],
                               sem_oa) for b in range(NDB)]

        def pass_zero(b, _):
            for t in range(BLK // 16):
                idx_v[b, pl.ds(t * 16, 16)] = rel[q][pl.ds(b * BLK + t * 16,
                                                           16)]
            pltpu.sync_copy(zbuf, acc_sh.at[idx_v.at[b]])
            return 0
        lax.fori_loop(0, nblk[q], pass_zero, 0)

        for d in zd:
            d.wait()
        plsc.subcore_barrier()

        def pass_add(b, _):
            pltpu.sync_copy(feat_hbm.at[pos[q].at[pl.ds(b * BLK, BLK)]],
                            feat0)
            pltpu.sync_copy(feat0, acc_sh.at[idx_v.at[b]], add=True)
            pltpu.sync_copy(ones16_v, cnt_sh.at[idx_v.at[b]], add=True)
            return 0
        lax.fori_loop(0, nblk[q], pass_add, 0)
        plsc.subcore_barrier()

        def dense_step(b, _):
            def step(cb, ab, nb, sa, sc, se, so,
                     cb2, ab2, nb2, sa2, sc2, se2, so2):
                @pl.when(jnp.logical_and(b >= 2, b < NDB))
                def _retire_out():
                    pltpu.make_async_copy(
                        cb, out_hbm.at[pl.ds(base + dense_rb(b - 2), BLK)],
                        so).wait()

                @pl.when(b < NDB)
                def _fetch():
                    rb = dense_rb(b)
                    pltpu.async_copy(ctr_hbm.at[pl.ds(base + rb, BLK)], cb, sa)
                    pltpu.async_copy(acc_sh.at[pl.ds(rb, BLK)], ab, sc)
                    pltpu.async_copy(cnt_sh.at[pl.ds(rb, BLK)], nb, se)

                @pl.when(b >= 1)
                def _combine_prev():
                    rbp = dense_rb(b - 1)
                    pltpu.make_async_copy(
                        ctr_hbm.at[pl.ds(base + rbp, BLK)], cb2, sa2).wait()
                    pltpu.make_async_copy(
                        acc_sh.at[pl.ds(rbp, BLK)], ab2, sc2).wait()
                    pltpu.make_async_copy(
                        cnt_sh.at[pl.ds(rbp, BLK)], nb2, se2).wait()

                    def combine(r, _):
                        cnt = nb2[r, pl.ds(0, 16)]
                        hit = cnt > 0.0
                        scale = 1.0 + A1 * cnt
                        for g in range(D // 16):
                            ctr = cb2[r, pl.ds(g * 16, 16)]
                            acc = ab2[r, pl.ds(g * 16, 16)]
                            cb2[r, pl.ds(g * 16, 16)] = jnp.where(
                                hit, ctr * scale - A1 * acc, ctr)
                        return 0
                    lax.fori_loop(0, BLK, combine, 0, unroll=4)
                    pltpu.async_copy(
                        cb2, out_hbm.at[pl.ds(base + rbp, BLK)], so2)

            @pl.when(b % 2 == 0)
            def _even():
                step(feat0, accb0, cntb0, sem_a, sem_c, sem_e, sem_oa,
                     feat1, accb1, cntb1, sem_b, sem_d, sem_f, sem_ob)

            @pl.when(b % 2 == 1)
            def _odd():
                step(feat1, accb1, cntb1, sem_b, sem_d, sem_f, sem_ob,
                     feat0, accb0, cntb0, sem_a, sem_c, sem_e, sem_oa)
            return 0
        lax.fori_loop(0, NDB + 1, dense_step, 0)

        pltpu.make_async_copy(
            ctrb[(NDB - 2) % 2],
            out_hbm.at[pl.ds(base + dense_rb(NDB - 2), BLK)],
            osem[(NDB - 2) % 2]).wait()
        pltpu.make_async_copy(
            ctrb[(NDB - 1) % 2],
            out_hbm.at[pl.ds(base + dense_rb(NDB - 1), BLK)],
            osem[(NDB - 1) % 2]).wait()

        plsc.subcore_barrier()


@jax.jit
def _run(features, labels, centers):
    mesh = plsc.VectorSubcoreMesh(core_axis_name="c", subcore_axis_name="s")
    lab2 = labels.reshape(NS, BT)
    o16 = jnp.ones((BLK, 16), jnp.float32)
    return pl.kernel(
        _body,
        out_type=jax.ShapeDtypeStruct((N_CENTER, D), jnp.float32),
        mesh=mesh,
        compiler_params=pltpu.CompilerParams(use_tc_tiling_on_sc=False, needs_layout_passes=False),
        scratch_types=[
            pltpu.VMEM((BT,), jnp.int32),
            pltpu.VMEM((16, BLK), jnp.int32),
            pltpu.VMEM((BLK, 16), jnp.float32),
            pltpu.VMEM((BLK, D), jnp.float32),
            pltpu.VMEM((BLK, 16), jnp.float32),
            pltpu.VMEM((LCAP,), jnp.int32),
            pltpu.VMEM((LCAP,), jnp.int32),
            pltpu.VMEM((LCAP,), jnp.int32),
            pltpu.VMEM((LCAP,), jnp.int32),
            pltpu.VMEM((LCAP,), jnp.int32),
            pltpu.VMEM((LCAP,), jnp.int32),
            pltpu.VMEM((BLK, D), jnp.float32),
            pltpu.VMEM((BLK, D), jnp.float32),
            pltpu.VMEM((BLK, D), jnp.float32),
            pltpu.VMEM((BLK, D), jnp.float32),
            pltpu.VMEM((BLK, 16), jnp.float32),
            pltpu.VMEM((BLK, 16), jnp.float32),
            pltpu.SemaphoreType.DMA,
            pltpu.SemaphoreType.DMA,
            pltpu.SemaphoreType.DMA,
            pltpu.SemaphoreType.DMA,
            pltpu.SemaphoreType.DMA,
            pltpu.SemaphoreType.DMA,
            pltpu.SemaphoreType.DMA,
            pltpu.SemaphoreType.DMA,
            pltpu.VMEM_SHARED((CHUNK + 256, D), jnp.float32),
            pltpu.VMEM_SHARED((CHUNK + 256, 16), jnp.float32),
        ],
    )(features, lab2, centers, o16)


def kernel(features, labels, centers):
    return _run(features, labels, centers)


# speedup vs baseline: 1.0065x; 1.0065x over previous
import jax
import jax.numpy as jnp
from jax import lax
from jax.experimental import pallas as pl
from jax.experimental.pallas import tpu as pltpu
from jax.experimental.pallas import tpu_sc as plsc

N_CENTER = 100000
D = 64
B = 16384
ALPHA = 0.9
A1 = ALPHA - 1.0

NC = 2
NS = 16
BT = B // NS
HALF = N_CENTER // NC
NCHUNK = 3
CHUNK = 16672
BLK = 64
NG = BT // 16
LCAP = BT + 16
TROWS = 1048
NDB = 17


def _body(feat_hbm, lab_hbm, ctr_hbm, o16_hbm, out_hbm,
          lab_v, idx_v, ones16_v, zbuf, cnt0,
          rel0, rel1, rel2, pos0, pos1, pos2,
          feat0, feat1, accb0, accb1, cntb0, cntb1,
          sem_a, sem_b, sem_c, sem_d, sem_e, sem_f, sem_oa, sem_ob,
          acc_sh, cnt_sh):
    c = lax.axis_index("c")
    s = lax.axis_index("s")

    pltpu.sync_copy(lab_hbm.at[s], lab_v)
    pltpu.sync_copy(o16_hbm, ones16_v)

    rel = [rel0, rel1, rel2]
    pos = [pos0, pos1, pos2]
    ctrb = [feat0, feat1]
    accb = [accb0, accb1]
    cntb = [cntb0, cntb1]
    asem = [sem_a, sem_b]
    bsem = [sem_c, sem_d]
    csem = [sem_e, sem_f]
    osem = [sem_oa, sem_ob]
    lanes = lax.iota(jnp.int32, 16)

    def fill_zb(i, _):
        zbuf[i // 4, pl.ds((i % 4) * 16, 16)] = jnp.zeros((16,), jnp.float32)
        return 0
    lax.fori_loop(0, BLK * 4, fill_zb, 0, unroll=4)

    def fill_c0(i, _):
        cnt0[i, pl.ds(0, 16)] = jnp.zeros((16,), jnp.float32)
        return 0
    lax.fori_loop(0, BLK, fill_c0, 0, unroll=4)

    nblk = []
    for q in range(NCHUNK):
        def prefill(i, _):
            rel[q][pl.ds(i * 16, 16)] = CHUNK + (i % 16) * 16 + lanes
            pos[q][pl.ds(i * 16, 16)] = jnp.zeros((16,), jnp.int32)
            return 0
        lax.fori_loop(0, LCAP // 16, prefill, 0, unroll=4)

        base = c * HALF + q * CHUNK

        def compact(i, off):
            v = lab_v[pl.ds(i * 16, 16)]
            r = v - base
            m = (r >= 0) & (r < CHUNK)
            mi = m.astype(jnp.int32)
            incl = plsc.cumsum(mi)
            dst = off + incl - mi
            plsc.store_scatter(rel[q], [dst], r, mask=m)
            p = s * BT + i * 16 + lanes
            plsc.store_scatter(pos[q], [dst], p, mask=m)
            return off + plsc.all_reduce_population_count(m)[0]
        off = lax.fori_loop(0, NG, compact, jnp.int32(0))
        nblk.append((off + (BLK - 1)) // BLK)

    for q in range(NCHUNK):
        base = c * HALF + q * CHUNK
        crows = min(CHUNK, HALF - q * CHUNK)
        tstart = jnp.minimum(s * TROWS, crows - TROWS)

        def dense_rb(b):
            return tstart + jnp.minimum(b * BLK, TROWS - BLK)

        zd = [pltpu.async_copy(cnt0, cnt_sh.at[pl.ds(dense_rb(b), BLK)],
                               sem_oa) for b in range(NDB)]

        def pass_zero(b, _):
            for t in range(BLK // 16):
                idx_v[b, pl.ds(t * 16, 16)] = rel[q][pl.ds(b * BLK + t * 16,
                                                           16)]
            pltpu.async_copy(zbuf, acc_sh.at[idx_v.at[b]], sem_ob)
            return 0
        lax.fori_loop(0, nblk[q], pass_zero, 0)

        for d in zd:
            d.wait()

        def drain_zs(b, _):
            pltpu.make_async_copy(zbuf, acc_sh.at[idx_v.at[b]], sem_ob).wait()
            return 0
        lax.fori_loop(0, nblk[q], drain_zs, 0)
        plsc.subcore_barrier()

        def pass_add(b, _):
            def do_block(buf, sem):
                @pl.when(b >= 2)
                def _retire():
                    pltpu.make_async_copy(
                        buf, acc_sh.at[idx_v.at[b - 2]], sem).wait()
                pltpu.sync_copy(feat_hbm.at[pos[q].at[pl.ds(b * BLK, BLK)]],
                                buf)
                pltpu.async_copy(buf, acc_sh.at[idx_v.at[b]], sem, add=True)
                pltpu.async_copy(ones16_v, cnt_sh.at[idx_v.at[b]], sem_ob,
                                 add=True)

            @pl.when(b % 2 == 0)
            def _even():
                do_block(feat0, sem_a)

            @pl.when(b % 2 == 1)
            def _odd():
                do_block(feat1, sem_b)
            return 0
        lax.fori_loop(0, nblk[q], pass_add, 0)

        def drain_sa(b, _):
            @pl.when(b % 2 == 0)
            def _even():
                pltpu.make_async_copy(
                    feat0, acc_sh.at[idx_v.at[b]], sem_a).wait()

            @pl.when(b % 2 == 1)
            def _odd():
                pltpu.make_async_copy(
                    feat1, acc_sh.at[idx_v.at[b]], sem_b).wait()
            return 0
        lax.fori_loop(jnp.maximum(nblk[q] - 2, 0), nblk[q], drain_sa, 0)

        def drain_ca(b, _):
            pltpu.make_async_copy(ones16_v, cnt_sh.at[idx_v.at[b]],
                                  sem_ob).wait()
            return 0
        lax.fori_loop(0, nblk[q], drain_ca, 0)
        plsc.subcore_barrier()

        def dense_step(b, _):
            def step(cb, ab, nb, sa, sc, se, so,
                     cb2, ab2, nb2, sa2, sc2, se2, so2):
                @pl.when(jnp.logical_and(b >= 2, b < NDB))
                def _retire_out():
                    pltpu.make_async_copy(
                        cb, out_hbm.at[pl.ds(base + dense_rb(b - 2), BLK)],
                        so).wait()

                @pl.when(b < NDB)
                def _fetch():
                    rb = dense_rb(b)
                    pltpu.async_copy(ctr_hbm.at[pl.ds(base + rb, BLK)], cb, sa)
                    pltpu.async_copy(acc_sh.at[pl.ds(rb, BLK)], ab, sc)
                    pltpu.async_copy(cnt_sh.at[pl.ds(rb, BLK)], nb, se)

                @pl.when(b >= 1)
                def _combine_prev():
                    rbp = dense_rb(b - 1)
                    pltpu.make_async_copy(
                        ctr_hbm.at[pl.ds(base + rbp, BLK)], cb2, sa2).wait()
                    pltpu.make_async_copy(
                        acc_sh.at[pl.ds(rbp, BLK)], ab2, sc2).wait()
                    pltpu.make_async_copy(
                        cnt_sh.at[pl.ds(rbp, BLK)], nb2, se2).wait()

                    def combine(r, _):
                        cnt = nb2[r, pl.ds(0, 16)]
                        hit = cnt > 0.0
                        scale = 1.0 + A1 * cnt
                        for g in range(D // 16):
                            ctr = cb2[r, pl.ds(g * 16, 16)]
                            acc = ab2[r, pl.ds(g * 16, 16)]
                            cb2[r, pl.ds(g * 16, 16)] = jnp.where(
                                hit, ctr * scale - A1 * acc, ctr)
                        return 0
                    lax.fori_loop(0, BLK, combine, 0, unroll=4)
                    pltpu.async_copy(
                        cb2, out_hbm.at[pl.ds(base + rbp, BLK)], so2)

            @pl.when(b % 2 == 0)
            def _even():
                step(feat0, accb0, cntb0, sem_a, sem_c, sem_e, sem_oa,
                     feat1, accb1, cntb1, sem_b, sem_d, sem_f, sem_ob)

            @pl.when(b % 2 == 1)
            def _odd():
                step(feat1, accb1, cntb1, sem_b, sem_d, sem_f, sem_ob,
                     feat0, accb0, cntb0, sem_a, sem_c, sem_e, sem_oa)
            return 0
        lax.fori_loop(0, NDB + 1, dense_step, 0)

        pltpu.make_async_copy(
            ctrb[(NDB - 2) % 2],
            out_hbm.at[pl.ds(base + dense_rb(NDB - 2), BLK)],
            osem[(NDB - 2) % 2]).wait()
        pltpu.make_async_copy(
            ctrb[(NDB - 1) % 2],
            out_hbm.at[pl.ds(base + dense_rb(NDB - 1), BLK)],
            osem[(NDB - 1) % 2]).wait()

        plsc.subcore_barrier()


@jax.jit
def _run(features, labels, centers):
    mesh = plsc.VectorSubcoreMesh(core_axis_name="c", subcore_axis_name="s")
    lab2 = labels.reshape(NS, BT)
    o16 = jnp.ones((BLK, 16), jnp.float32)
    return pl.kernel(
        _body,
        out_type=jax.ShapeDtypeStruct((N_CENTER, D), jnp.float32),
        mesh=mesh,
        compiler_params=pltpu.CompilerParams(use_tc_tiling_on_sc=False, needs_layout_passes=False),
        scratch_types=[
            pltpu.VMEM((BT,), jnp.int32),
            pltpu.VMEM((16, BLK), jnp.int32),
            pltpu.VMEM((BLK, 16), jnp.float32),
            pltpu.VMEM((BLK, D), jnp.float32),
            pltpu.VMEM((BLK, 16), jnp.float32),
            pltpu.VMEM((LCAP,), jnp.int32),
            pltpu.VMEM((LCAP,), jnp.int32),
            pltpu.VMEM((LCAP,), jnp.int32),
            pltpu.VMEM((LCAP,), jnp.int32),
            pltpu.VMEM((LCAP,), jnp.int32),
            pltpu.VMEM((LCAP,), jnp.int32),
            pltpu.VMEM((BLK, D), jnp.float32),
            pltpu.VMEM((BLK, D), jnp.float32),
            pltpu.VMEM((BLK, D), jnp.float32),
            pltpu.VMEM((BLK, D), jnp.float32),
            pltpu.VMEM((BLK, 16), jnp.float32),
            pltpu.VMEM((BLK, 16), jnp.float32),
            pltpu.SemaphoreType.DMA,
            pltpu.SemaphoreType.DMA,
            pltpu.SemaphoreType.DMA,
            pltpu.SemaphoreType.DMA,
            pltpu.SemaphoreType.DMA,
            pltpu.SemaphoreType.DMA,
            pltpu.SemaphoreType.DMA,
            pltpu.SemaphoreType.DMA,
            pltpu.VMEM_SHARED((CHUNK + 256, D), jnp.float32),
            pltpu.VMEM_SHARED((CHUNK + 256, 16), jnp.float32),
        ],
    )(features, lab2, centers, o16)


def kernel(features, labels, centers):
    return _run(features, labels, centers)


# speedup vs baseline: 1.0716x; 1.0647x over previous
import jax
import jax.numpy as jnp
from jax import lax
from jax.experimental import pallas as pl
from jax.experimental.pallas import tpu as pltpu
from jax.experimental.pallas import tpu_sc as plsc

N_CENTER = 100000
D = 64
B = 16384
ALPHA = 0.9
A1 = ALPHA - 1.0

NC = 2
NS = 16
BT = B // NS
HALF = N_CENTER // NC
NCHUNK = 3
CHUNK = 16672
BLK = 64
NJ = BT // BLK
TROWS = 1048
NDB = 17


def _body(feat_hbm, lab_hbm, ctr_hbm, o16_hbm, out_hbm,
          ones16_v, lab_v, idx_v,
          feat0, feat1, accb0, accb1, cnt0, cnt1,
          sem_a, sem_b, sem_c, sem_d, sem_e, sem_f, sem_oa, sem_ob,
          acc_sh, cnt_sh):
    c = lax.axis_index("c")
    s = lax.axis_index("s")

    pltpu.sync_copy(lab_hbm.at[s], lab_v)
    pltpu.sync_copy(o16_hbm, ones16_v)

    feat = [feat0, feat1]
    ctrb = [feat0, feat1]
    accb = [accb0, accb1]
    cntb = [cnt0, cnt1]
    asem = [sem_a, sem_b]
    bsem = [sem_c, sem_d]
    csem = [sem_e, sem_f]
    osem = [sem_oa, sem_ob]

    def chunk_body(chunk, _):
        base = c * HALF + chunk * CHUNK
        crows = jnp.minimum(CHUNK, HALF - chunk * CHUNK)
        tstart = jnp.minimum(s * TROWS, crows - TROWS)

        def dense_rb(b):
            return tstart + min(b * BLK, TROWS - BLK)

        def fill_zero(i, _):
            feat0[i // 4, pl.ds((i % 4) * 16, 16)] = jnp.zeros(
                (16,), jnp.float32)
            return 0
        lax.fori_loop(0, BLK * 4, fill_zero, 0, unroll=4)

        def fill_zero16(i, _):
            cnt0[i, pl.ds(0, 16)] = jnp.zeros((16,), jnp.float32)
            return 0
        lax.fori_loop(0, BLK, fill_zero16, 0, unroll=4)

        zd = []
        for b in range(NDB):
            rb = dense_rb(b)
            zd.append(pltpu.async_copy(
                feat0, acc_sh.at[pl.ds(rb, BLK)], sem_oa))
            zd.append(pltpu.async_copy(
                cnt0, cnt_sh.at[pl.ds(rb, BLK)], sem_ob))
        for d in zd:
            d.wait()
        plsc.subcore_barrier()

        scat = [None, None]

        def fetch(j):
            sl = j % 2
            return pltpu.async_copy(
                feat_hbm.at[pl.ds(s * BT + j * BLK, BLK)], feat[sl], asem[sl])

        pend = fetch(0)
        for j in range(NJ):
            sl = j % 2
            nxt = None
            if j + 1 < NJ:
                if scat[(j + 1) % 2] is not None:
                    scat[(j + 1) % 2][0].wait()
                    scat[(j + 1) % 2][1].wait()
                    scat[(j + 1) % 2] = None
                nxt = fetch(j + 1)
            pend.wait()

            for k in range(BLK // 16):
                v = lab_v[j, pl.ds(k * 16, 16)]
                rel = v - base
                inb = (rel >= 0) & (rel < CHUNK)
                dummy = CHUNK + ((j * 4 + k) % 16) * 16 + lax.iota(
                    jnp.int32, 16)
                idx_v[j, pl.ds(k * 16, 16)] = jnp.where(inb, rel, dummy)

            scat[sl] = (
                pltpu.async_copy(feat[sl], acc_sh.at[idx_v.at[j]],
                                 osem[sl], add=True),
                pltpu.async_copy(ones16_v, cnt_sh.at[idx_v.at[j]],
                                 csem[sl], add=True),
            )
            if nxt is not None:
                pend = nxt
        for d in scat:
            if d is not None:
                d[0].wait()
                d[1].wait()
        plsc.subcore_barrier()

        owr = [None, None]

        def issue_dense(b):
            sl = b % 2
            rb = dense_rb(b)
            return (
                pltpu.async_copy(ctr_hbm.at[pl.ds(base + rb, BLK)],
                                 ctrb[sl], asem[sl]),
                pltpu.async_copy(acc_sh.at[pl.ds(rb, BLK)], accb[sl],
                                 bsem[sl]),
                pltpu.async_copy(cnt_sh.at[pl.ds(rb, BLK)], cntb[sl],
                                 csem[sl]),
            )

        pend = issue_dense(0)
        for b in range(NDB):
            sl = b % 2
            nxt = None
            if b + 1 < NDB:
                if owr[(b + 1) % 2] is not None:
                    owr[(b + 1) % 2].wait()
                    owr[(b + 1) % 2] = None
                nxt = issue_dense(b + 1)
            pend[0].wait()
            pend[1].wait()
            pend[2].wait()

            def combine(r, _):
                cnt = cntb[sl][r, pl.ds(0, 16)]
                scale = 1.0 + A1 * cnt
                for g in range(D // 16):
                    ctr = ctrb[sl][r, pl.ds(g * 16, 16)]
                    acc = accb[sl][r, pl.ds(g * 16, 16)]
                    ctrb[sl][r, pl.ds(g * 16, 16)] = ctr * scale - A1 * acc
                return 0
            lax.fori_loop(0, BLK, combine, 0, unroll=4)

            owr[sl] = pltpu.async_copy(
                ctrb[sl], out_hbm.at[pl.ds(base + dense_rb(b), BLK)], osem[sl])
            if nxt is not None:
                pend = nxt
        for d in owr:
            if d is not None:
                d.wait()

        plsc.subcore_barrier()
        return 0

    lax.fori_loop(0, NCHUNK, chunk_body, 0)


@jax.jit
def _run(features, labels, centers):
    mesh = plsc.VectorSubcoreMesh(core_axis_name="c", subcore_axis_name="s")
    lab3 = labels.reshape(NS, NJ, BLK)
    o16 = jnp.ones((BLK, 16), jnp.float32)
    return pl.kernel(
        _body,
        out_type=jax.ShapeDtypeStruct((N_CENTER, D), jnp.float32),
        mesh=mesh,
        compiler_params=pltpu.CompilerParams(use_tc_tiling_on_sc=False),
        scratch_types=[
            pltpu.VMEM((BLK, 16), jnp.float32),
            pltpu.VMEM((NJ, BLK), jnp.int32),
            pltpu.VMEM((NJ, BLK), jnp.int32),
            pltpu.VMEM((BLK, D), jnp.float32),
            pltpu.VMEM((BLK, D), jnp.float32),
            pltpu.VMEM((BLK, D), jnp.float32),
            pltpu.VMEM((BLK, D), jnp.float32),
            pltpu.VMEM((BLK, 16), jnp.float32),
            pltpu.VMEM((BLK, 16), jnp.float32),
            pltpu.SemaphoreType.DMA,
            pltpu.SemaphoreType.DMA,
            pltpu.SemaphoreType.DMA,
            pltpu.SemaphoreType.DMA,
            pltpu.SemaphoreType.DMA,
            pltpu.SemaphoreType.DMA,
            pltpu.SemaphoreType.DMA,
            pltpu.SemaphoreType.DMA,
            pltpu.VMEM_SHARED((CHUNK + 256, D), jnp.float32),
            pltpu.VMEM_SHARED((CHUNK + 256, 16), jnp.float32),
        ],
    )(features, lab3, centers, o16)


def kernel(features, labels, centers):
    return _run(features, labels, centers)


# speedup vs baseline: 1.0768x; 1.0049x over previous
import jax
import jax.numpy as jnp
from jax import lax
from jax.experimental import pallas as pl
from jax.experimental.pallas import tpu as pltpu
from jax.experimental.pallas import tpu_sc as plsc

N_CENTER = 100000
D = 64
B = 16384
ALPHA = 0.9
A1 = ALPHA - 1.0

NC = 2
NS = 16
BT = B // NS
HALF = N_CENTER // NC
NCHUNK = 3
CHUNK = 16672
BLK = 64
NJ = BT // BLK
TROWS = 1048
NDB = 17


def _body(feat_hbm, lab_hbm, ctr_hbm, o16_hbm, out_hbm,
          ones16_v, lab_v, idx_v, zbuf, zcnt,
          feat0, feat1, accb0, accb1, cnt0, cnt1,
          sem_a, sem_b, sem_c, sem_d, sem_e, sem_f, sem_oa, sem_ob,
          acc_sh, cnt_sh):
    c = lax.axis_index("c")
    s = lax.axis_index("s")

    pltpu.sync_copy(lab_hbm.at[s], lab_v)
    pltpu.sync_copy(o16_hbm, ones16_v)

    feat = [feat0, feat1]
    ctrb = [feat0, feat1]
    accb = [accb0, accb1]
    cntb = [cnt0, cnt1]
    asem = [sem_a, sem_b]
    bsem = [sem_c, sem_d]
    csem = [sem_e, sem_f]
    osem = [sem_oa, sem_ob]

    def fill_zero(i, _):
        zbuf[i // 4, pl.ds((i % 4) * 16, 16)] = jnp.zeros((16,), jnp.float32)
        return 0
    lax.fori_loop(0, BLK * 4, fill_zero, 0, unroll=4)

    def fill_zero16(i, _):
        zcnt[i, pl.ds(0, 16)] = jnp.zeros((16,), jnp.float32)
        return 0
    lax.fori_loop(0, BLK, fill_zero16, 0, unroll=4)

    def chunk_body(chunk, _):
        base = c * HALF + chunk * CHUNK
        crows = jnp.minimum(CHUNK, HALF - chunk * CHUNK)
        tstart = jnp.minimum(s * TROWS, crows - TROWS)

        def dense_rb(b):
            return tstart + min(b * BLK, TROWS - BLK)

        zd = []
        for b in range(NDB):
            rb = dense_rb(b)
            zd.append(pltpu.async_copy(
                zbuf, acc_sh.at[pl.ds(rb, BLK)], sem_oa))
            zd.append(pltpu.async_copy(
                zcnt, cnt_sh.at[pl.ds(rb, BLK)], sem_ob))
        for d in zd:
            d.wait()
        plsc.subcore_barrier()

        scat = [None, None]

        def fetch(j):
            sl = j % 2
            return pltpu.async_copy(
                feat_hbm.at[pl.ds(s * BT + j * BLK, BLK)], feat[sl], asem[sl])

        pend = fetch(0)
        for j in range(NJ):
            sl = j % 2
            nxt = None
            if j + 1 < NJ:
                if scat[(j + 1) % 2] is not None:
                    scat[(j + 1) % 2][0].wait()
                    scat[(j + 1) % 2][1].wait()
                    scat[(j + 1) % 2] = None
                nxt = fetch(j + 1)
            pend.wait()

            for k in range(BLK // 16):
                v = lab_v[j, pl.ds(k * 16, 16)]
                rel = v - base
                inb = (rel >= 0) & (rel < CHUNK)
                dummy = CHUNK + ((j * 4 + k) % 16) * 16 + lax.iota(
                    jnp.int32, 16)
                idx_v[j, pl.ds(k * 16, 16)] = jnp.where(inb, rel, dummy)

            scat[sl] = (
                pltpu.async_copy(feat[sl], acc_sh.at[idx_v.at[j]],
                                 osem[sl], add=True),
                pltpu.async_copy(ones16_v, cnt_sh.at[idx_v.at[j]],
                                 csem[sl], add=True),
            )
            if nxt is not None:
                pend = nxt
        for d in scat:
            if d is not None:
                d[0].wait()
                d[1].wait()
        plsc.subcore_barrier()

        owr = [None, None]

        def issue_dense(b):
            sl = b % 2
            rb = dense_rb(b)
            return (
                pltpu.async_copy(ctr_hbm.at[pl.ds(base + rb, BLK)],
                                 ctrb[sl], asem[sl]),
                pltpu.async_copy(acc_sh.at[pl.ds(rb, BLK)], accb[sl],
                                 bsem[sl]),
                pltpu.async_copy(cnt_sh.at[pl.ds(rb, BLK)], cntb[sl],
                                 csem[sl]),
            )

        pend = issue_dense(0)
        for b in range(NDB):
            sl = b % 2
            nxt = None
            if b + 1 < NDB:
                if owr[(b + 1) % 2] is not None:
                    owr[(b + 1) % 2].wait()
                    owr[(b + 1) % 2] = None
                nxt = issue_dense(b + 1)
            pend[0].wait()
            pend[1].wait()
            pend[2].wait()

            def combine(r, _):
                cnt = cntb[sl][r, pl.ds(0, 16)]
                scale = 1.0 + A1 * cnt
                for g in range(D // 16):
                    ctr = ctrb[sl][r, pl.ds(g * 16, 16)]
                    acc = accb[sl][r, pl.ds(g * 16, 16)]
                    ctrb[sl][r, pl.ds(g * 16, 16)] = ctr * scale - A1 * acc
                return 0
            lax.fori_loop(0, BLK, combine, 0, unroll=8)

            owr[sl] = pltpu.async_copy(
                ctrb[sl], out_hbm.at[pl.ds(base + dense_rb(b), BLK)], osem[sl])
            if nxt is not None:
                pend = nxt
        for d in owr:
            if d is not None:
                d.wait()

        plsc.subcore_barrier()
        return 0

    lax.fori_loop(0, NCHUNK, chunk_body, 0)


@jax.jit
def _run(features, labels, centers):
    mesh = plsc.VectorSubcoreMesh(core_axis_name="c", subcore_axis_name="s")
    lab3 = labels.reshape(NS, NJ, BLK)
    o16 = jnp.ones((BLK, 16), jnp.float32)
    return pl.kernel(
        _body,
        out_type=jax.ShapeDtypeStruct((N_CENTER, D), jnp.float32),
        mesh=mesh,
        compiler_params=pltpu.CompilerParams(use_tc_tiling_on_sc=False),
        scratch_types=[
            pltpu.VMEM((BLK, 16), jnp.float32),
            pltpu.VMEM((NJ, BLK), jnp.int32),
            pltpu.VMEM((NJ, BLK), jnp.int32),
            pltpu.VMEM((BLK, D), jnp.float32),
            pltpu.VMEM((BLK, 16), jnp.float32),
            pltpu.VMEM((BLK, D), jnp.float32),
            pltpu.VMEM((BLK, D), jnp.float32),
            pltpu.VMEM((BLK, D), jnp.float32),
            pltpu.VMEM((BLK, D), jnp.float32),
            pltpu.VMEM((BLK, 16), jnp.float32),
            pltpu.VMEM((BLK, 16), jnp.float32),
            pltpu.SemaphoreType.DMA,
            pltpu.SemaphoreType.DMA,
            pltpu.SemaphoreType.DMA,
            pltpu.SemaphoreType.DMA,
            pltpu.SemaphoreType.DMA,
            pltpu.SemaphoreType.DMA,
            pltpu.SemaphoreType.DMA,
            pltpu.SemaphoreType.DMA,
            pltpu.VMEM_SHARED((CHUNK + 256, D), jnp.float32),
            pltpu.VMEM_SHARED((CHUNK + 256, 16), jnp.float32),
        ],
    )(features, lab3, centers, o16)


def kernel(features, labels, centers):
    return _run(features, labels, centers)


# speedup vs baseline: 1.0806x; 1.0036x over previous
import jax
import jax.numpy as jnp
from jax import lax
from jax.experimental import pallas as pl
from jax.experimental.pallas import tpu as pltpu
from jax.experimental.pallas import tpu_sc as plsc

N_CENTER = 100000
D = 64
B = 16384
ALPHA = 0.9
A1 = ALPHA - 1.0

NC = 2
NS = 16
BT = B // NS
HALF = N_CENTER // NC
NCHUNK = 3
CHUNK = 16672
BLK = 64
NJ = BT // BLK
TROWS = 1048
NDB = 17


def _body(feat_hbm, lab_hbm, ctr_hbm, o16_hbm, out_hbm,
          ones16_v, lab_v, idx_v, zbuf, zcnt,
          feat0, feat1, accb0, accb1, cnt0, cnt1,
          sem_a, sem_b, sem_c, sem_d, sem_e, sem_f, sem_oa, sem_ob,
          acc_sh, cnt_sh):
    c = lax.axis_index("c")
    s = lax.axis_index("s")

    pltpu.sync_copy(lab_hbm.at[s], lab_v)
    pltpu.sync_copy(o16_hbm, ones16_v)

    feat = [feat0, feat1]
    ctrb = [feat0, feat1]
    accb = [accb0, accb1]
    cntb = [cnt0, cnt1]
    asem = [sem_a, sem_b]
    bsem = [sem_c, sem_d]
    csem = [sem_e, sem_f]
    osem = [sem_oa, sem_ob]

    def fill_zero(i, _):
        zbuf[i // 4, pl.ds((i % 4) * 16, 16)] = jnp.zeros((16,), jnp.float32)
        return 0
    lax.fori_loop(0, BLK * 4, fill_zero, 0, unroll=4)

    def fill_zero16(i, _):
        zcnt[i, pl.ds(0, 16)] = jnp.zeros((16,), jnp.float32)
        return 0
    lax.fori_loop(0, BLK, fill_zero16, 0, unroll=4)

    def chunk_body(chunk, _):
        base = c * HALF + chunk * CHUNK
        crows = jnp.minimum(CHUNK, HALF - chunk * CHUNK)
        tstart = jnp.minimum(s * TROWS, crows - TROWS)

        def dense_rb(b):
            return tstart + min(b * BLK, TROWS - BLK)

        zd = []
        for b in range(NDB):
            rb = dense_rb(b)
            zd.append(pltpu.async_copy(
                zbuf, acc_sh.at[pl.ds(rb, BLK)], sem_oa))
            zd.append(pltpu.async_copy(
                zcnt, cnt_sh.at[pl.ds(rb, BLK)], sem_ob))
        for j in range(NJ):
            for k in range(BLK // 16):
                v = lab_v[j, pl.ds(k * 16, 16)]
                rel = v - base
                inb = (rel >= 0) & (rel < CHUNK)
                dummy = CHUNK + ((j * 4 + k) % 16) * 16 + lax.iota(
                    jnp.int32, 16)
                idx_v[j, pl.ds(k * 16, 16)] = jnp.where(inb, rel, dummy)

        def fetch(j):
            sl = j % 2
            return pltpu.async_copy(
                feat_hbm.at[pl.ds(s * BT + j * BLK, BLK)], feat[sl], asem[sl])

        pend = fetch(0)
        for d in zd:
            d.wait()
        plsc.subcore_barrier()

        scat = [None, None]
        for j in range(NJ):
            sl = j % 2
            nxt = None
            if j + 1 < NJ:
                if scat[(j + 1) % 2] is not None:
                    scat[(j + 1) % 2][0].wait()
                    scat[(j + 1) % 2][1].wait()
                    scat[(j + 1) % 2] = None
                nxt = fetch(j + 1)
            pend.wait()

            scat[sl] = (
                pltpu.async_copy(feat[sl], acc_sh.at[idx_v.at[j]],
                                 osem[sl], add=True),
                pltpu.async_copy(ones16_v, cnt_sh.at[idx_v.at[j]],
                                 csem[sl], add=True),
            )
            if nxt is not None:
                pend = nxt
        for d in scat:
            if d is not None:
                d[0].wait()
                d[1].wait()
        plsc.subcore_barrier()

        owr = [None, None]

        def issue_dense(b):
            sl = b % 2
            rb = dense_rb(b)
            return (
                pltpu.async_copy(ctr_hbm.at[pl.ds(base + rb, BLK)],
                                 ctrb[sl], asem[sl]),
                pltpu.async_copy(acc_sh.at[pl.ds(rb, BLK)], accb[sl],
                                 bsem[sl]),
                pltpu.async_copy(cnt_sh.at[pl.ds(rb, BLK)], cntb[sl],
                                 csem[sl]),
            )

        pend = issue_dense(0)
        for b in range(NDB):
            sl = b % 2
            nxt = None
            if b + 1 < NDB:
                if owr[(b + 1) % 2] is not None:
                    owr[(b + 1) % 2].wait()
                    owr[(b + 1) % 2] = None
                nxt = issue_dense(b + 1)
            pend[0].wait()
            pend[1].wait()
            pend[2].wait()

            def combine(r, _):
                cnt = cntb[sl][r, pl.ds(0, 16)]
                scale = 1.0 + A1 * cnt
                for g in range(D // 16):
                    ctr = ctrb[sl][r, pl.ds(g * 16, 16)]
                    acc = accb[sl][r, pl.ds(g * 16, 16)]
                    ctrb[sl][r, pl.ds(g * 16, 16)] = ctr * scale - A1 * acc
                return 0
            lax.fori_loop(0, BLK, combine, 0, unroll=8)

            owr[sl] = pltpu.async_copy(
                ctrb[sl], out_hbm.at[pl.ds(base + dense_rb(b), BLK)], osem[sl])
            if nxt is not None:
                pend = nxt
        for d in owr:
            if d is not None:
                d.wait()

        plsc.subcore_barrier()
        return 0

    lax.fori_loop(0, NCHUNK, chunk_body, 0)


@jax.jit
def _run(features, labels, centers):
    mesh = plsc.VectorSubcoreMesh(core_axis_name="c", subcore_axis_name="s")
    lab3 = labels.reshape(NS, NJ, BLK)
    o16 = jnp.ones((BLK, 16), jnp.float32)
    return pl.kernel(
        _body,
        out_type=jax.ShapeDtypeStruct((N_CENTER, D), jnp.float32),
        mesh=mesh,
        compiler_params=pltpu.CompilerParams(use_tc_tiling_on_sc=False),
        scratch_types=[
            pltpu.VMEM((BLK, 16), jnp.float32),
            pltpu.VMEM((NJ, BLK), jnp.int32),
            pltpu.VMEM((NJ, BLK), jnp.int32),
            pltpu.VMEM((BLK, D), jnp.float32),
            pltpu.VMEM((BLK, 16), jnp.float32),
            pltpu.VMEM((BLK, D), jnp.float32),
            pltpu.VMEM((BLK, D), jnp.float32),
            pltpu.VMEM((BLK, D), jnp.float32),
            pltpu.VMEM((BLK, D), jnp.float32),
            pltpu.VMEM((BLK, 16), jnp.float32),
            pltpu.VMEM((BLK, 16), jnp.float32),
            pltpu.SemaphoreType.DMA,
            pltpu.SemaphoreType.DMA,
            pltpu.SemaphoreType.DMA,
            pltpu.SemaphoreType.DMA,
            pltpu.SemaphoreType.DMA,
            pltpu.SemaphoreType.DMA,
            pltpu.SemaphoreType.DMA,
            pltpu.SemaphoreType.DMA,
            pltpu.VMEM_SHARED((CHUNK + 256, D), jnp.float32),
            pltpu.VMEM_SHARED((CHUNK + 256, 16), jnp.float32),
        ],
    )(features, lab3, centers, o16)


def kernel(features, labels, centers):
    return _run(features, labels, centers)
